# baseline (speedup 1.0000x reference)
import functools

import jax
import jax.numpy as jnp
from jax import lax
from jax.experimental import pallas as pl
from jax.experimental.pallas import tpu as pltpu
from jax.experimental.pallas import tpu_sc as plsc

VOCAB = 100000
EMBED = 768
CTX = 2048
B = 4
S = 2048

NUM_CORES = 2
NUM_SUBCORES = 16
NUM_WORKERS = NUM_CORES * NUM_SUBCORES
S_BLK = S // NUM_WORKERS
S_CHUNK = 16
NCHUNK = S_BLK // S_CHUNK
ROWS = B * S_CHUNK
LANES = 16
COL_CHUNKS = EMBED // LANES


def _emb_kernel(idx_hbm, tok_hbm, pos_hbm, out_hbm, idx_v, pos_v, rows_v,
                gsem, wsem):
    wid = lax.axis_index("s") * NUM_CORES + lax.axis_index("c")
    s0 = wid * S_BLK

    pltpu.sync_copy(pos_hbm.at[pl.ds(s0, S_BLK)], pos_v)
    pltpu.sync_copy(idx_hbm.at[wid], idx_v)

    writes = []
    for c in range(NCHUNK):
        for w in writes:
            w.wait()
        writes = []
        pltpu.async_copy(tok_hbm.at[idx_v.at[c]], rows_v, gsem).wait()

        def s_body(t, carry):
            for j in range(COL_CHUNKS):
                sl = pl.ds(j * LANES, LANES)
                p = pos_v[c * S_CHUNK + t, sl]
                for b in range(B):
                    plsc.addupdate(rows_v.at[b * S_CHUNK + t, sl], p)
            return carry

        lax.fori_loop(0, S_CHUNK, s_body, 0)

        for b in range(B):
            base = b * S + s0 + c * S_CHUNK
            writes.append(
                pltpu.async_copy(rows_v.at[pl.ds(b * S_CHUNK, S_CHUNK)],
                                 out_hbm.at[pl.ds(base, S_CHUNK)], wsem))
    for w in writes:
        w.wait()


@jax.jit
def _run(idx_re, token_table, pos_table):
    mesh = plsc.VectorSubcoreMesh(core_axis_name="c", subcore_axis_name="s")
    f = functools.partial(
        pl.kernel,
        mesh=mesh,
        out_type=jax.ShapeDtypeStruct((B * S, EMBED), jnp.float32),
        scratch_types=[
            pltpu.VMEM((NCHUNK, ROWS), jnp.int32),
            pltpu.VMEM((S_BLK, EMBED), jnp.float32),
            pltpu.VMEM((ROWS, EMBED), jnp.float32),
            pltpu.SemaphoreType.DMA,
            pltpu.SemaphoreType.DMA,
        ],
    )(_emb_kernel)
    return f(idx_re, token_table, pos_table)


def kernel(tok_idx, token_table, pos_table):
    idx_re = jnp.transpose(
        tok_idx.astype(jnp.int32).reshape(B, NUM_WORKERS, NCHUNK, S_CHUNK),
        (1, 2, 0, 3)).reshape(NUM_WORKERS, NCHUNK, ROWS)
    out = _run(idx_re, token_table, pos_table)
    return out.reshape(B, S, EMBED)
